# Initial kernel scaffold; baseline (speedup 1.0000x reference)
import jax
import jax.numpy as jnp
from jax.experimental import pallas as pl


def kernel(x, edge_index, edge_attr, batch, en1_W1, en1_b1, en1_W2, en1_b2, en2_W1, en2_b1, en2_W2, en2_b2, root1, bias1, root2, bias2, fc1_W, fc1_b, fc2_W, fc2_b, fc3_W, fc3_b, fc4_W, fc4_b):
    raise NotImplementedError("write your pallas kernel here")


# speedup vs baseline: 2.1537x; 2.1537x over previous
import functools

import jax
import jax.numpy as jnp
from jax import lax
from jax.experimental import pallas as pl
from jax.experimental.pallas import tpu as pltpu
from jax.experimental.pallas import tpu_sc as plsc

N = 10000
E = 160000
IN_C = 32
HID = 16
N_CLASSES = 10
N_GRAPHS = 64

NC = 2
NS = 16
CHUNK = 128
CHUNKS_PER_TILE = 40
E_TILE = CHUNK * CHUNKS_PER_TILE
EP = E_TILE * NC * NS
NROWS = N + 16
ZROWS = NROWS // NS
OROWS = N // NS

EB = 640
N_EBLK = EP // EB


def _mesh():
    return plsc.VectorSubcoreMesh(core_axis_name="c", subcore_axis_name="s")


_SC_PARAMS = pltpu.CompilerParams(use_tc_tiling_on_sc=False)


def _gather_body(d, table_hbm, idx2d_hbm, out_hbm, idx_v, rows_v, sem):
    c = lax.axis_index("c")
    s = lax.axis_index("s")
    wid = s * NC + c
    pltpu.sync_copy(idx2d_hbm.at[pl.ds(wid * CHUNKS_PER_TILE, CHUNKS_PER_TILE)],
                    idx_v)

    def body(j, carry):
        pltpu.async_copy(table_hbm.at[idx_v.at[j]], rows_v, sem).wait()
        pltpu.sync_copy(rows_v,
                        out_hbm.at[pl.ds(wid * E_TILE + j * CHUNK, CHUNK)])
        return carry

    lax.fori_loop(0, CHUNKS_PER_TILE, body, 0)


def _make_gather(d):
    return functools.partial(
        pl.kernel,
        out_type=jax.ShapeDtypeStruct((EP, d), jnp.float32),
        mesh=_mesh(),
        compiler_params=_SC_PARAMS,
        scratch_types=[
            pltpu.VMEM((CHUNKS_PER_TILE, CHUNK), jnp.int32),
            pltpu.VMEM((CHUNK, d), jnp.float32),
            pltpu.SemaphoreType.DMA,
        ],
    )(functools.partial(_gather_body, d))


def _scatter1_body(msg_hbm, idx2d_hbm, msg_out, cnt_out,
                   idx_v, msg_buf, ones_buf, zrow, tab_msg, tab_cnt):
    c = lax.axis_index("c")
    s = lax.axis_index("s")
    wid = s * NC + c

    z16 = jnp.zeros((16,), jnp.float32)
    o16 = jnp.ones((16,), jnp.float32)

    def fill_z(i, carry):
        zrow[i, :] = z16
        return carry

    lax.fori_loop(0, ZROWS, fill_z, 0)

    def fill_o(i, carry):
        ones_buf[i, :] = o16
        return carry

    lax.fori_loop(0, CHUNK, fill_o, 0)

    pltpu.sync_copy(zrow, tab_msg.at[pl.ds(s * ZROWS, ZROWS)])
    pltpu.sync_copy(zrow, tab_cnt.at[pl.ds(s * ZROWS, ZROWS)])
    plsc.subcore_barrier()

    pltpu.sync_copy(idx2d_hbm.at[pl.ds(wid * CHUNKS_PER_TILE, CHUNKS_PER_TILE)],
                    idx_v)

    def body(j, carry):
        pltpu.sync_copy(msg_hbm.at[pl.ds(wid * E_TILE + j * CHUNK, CHUNK)],
                        msg_buf)
        pltpu.sync_copy(msg_buf, tab_msg.at[idx_v.at[j]], add=True)
        pltpu.sync_copy(ones_buf, tab_cnt.at[idx_v.at[j]], add=True)
        return carry

    lax.fori_loop(0, CHUNKS_PER_TILE, body, 0)
    plsc.subcore_barrier()

    pltpu.sync_copy(tab_msg.at[pl.ds(s * OROWS, OROWS)],
                    msg_out.at[c, pl.ds(s * OROWS, OROWS)])
    pltpu.sync_copy(tab_cnt.at[pl.ds(s * OROWS, OROWS)],
                    cnt_out.at[c, pl.ds(s * OROWS, OROWS)])


def _make_scatter1():
    return functools.partial(
        pl.kernel,
        out_type=[jax.ShapeDtypeStruct((NC, N, HID), jnp.float32),
                  jax.ShapeDtypeStruct((NC, N, HID), jnp.float32)],
        mesh=_mesh(),
        compiler_params=_SC_PARAMS,
        scratch_types=[
            pltpu.VMEM((CHUNKS_PER_TILE, CHUNK), jnp.int32),
            pltpu.VMEM((CHUNK, HID), jnp.float32),
            pltpu.VMEM((CHUNK, HID), jnp.float32),
            pltpu.VMEM((ZROWS, HID), jnp.float32),
            pltpu.VMEM_SHARED((NROWS, HID), jnp.float32),
            pltpu.VMEM_SHARED((NROWS, HID), jnp.float32),
        ],
    )(_scatter1_body)


def _scatter2_body(msg_hbm, idx2d_hbm, msg_out, idx_v, msg_buf, zrow, tab_msg):
    c = lax.axis_index("c")
    s = lax.axis_index("s")
    wid = s * NC + c

    z16 = jnp.zeros((16,), jnp.float32)

    def fill_z(i, carry):
        zrow[i, 0:16] = z16
        zrow[i, 16:32] = z16
        return carry

    lax.fori_loop(0, ZROWS, fill_z, 0)

    pltpu.sync_copy(zrow, tab_msg.at[pl.ds(s * ZROWS, ZROWS)])
    plsc.subcore_barrier()

    pltpu.sync_copy(idx2d_hbm.at[pl.ds(wid * CHUNKS_PER_TILE, CHUNKS_PER_TILE)],
                    idx_v)

    def body(j, carry):
        pltpu.sync_copy(msg_hbm.at[pl.ds(wid * E_TILE + j * CHUNK, CHUNK)],
                        msg_buf)
        pltpu.sync_copy(msg_buf, tab_msg.at[idx_v.at[j]], add=True)
        return carry

    lax.fori_loop(0, CHUNKS_PER_TILE, body, 0)
    plsc.subcore_barrier()

    pltpu.sync_copy(tab_msg.at[pl.ds(s * OROWS, OROWS)],
                    msg_out.at[c, pl.ds(s * OROWS, OROWS)])


def _make_scatter2():
    return functools.partial(
        pl.kernel,
        out_type=jax.ShapeDtypeStruct((NC, N, 2 * HID), jnp.float32),
        mesh=_mesh(),
        compiler_params=_SC_PARAMS,
        scratch_types=[
            pltpu.VMEM((CHUNKS_PER_TILE, CHUNK), jnp.int32),
            pltpu.VMEM((CHUNK, 2 * HID), jnp.float32),
            pltpu.VMEM((ZROWS, 2 * HID), jnp.float32),
            pltpu.VMEM_SHARED((NROWS, 2 * HID), jnp.float32),
        ],
    )(_scatter2_body)


def _unpack_bits_block(ea_i32):
    ea_f = ea_i32.astype(jnp.float32)
    b0 = jnp.broadcast_to(ea_f[:, 0:1], (EB, 8))
    b1 = jnp.broadcast_to(ea_f[:, 1:2], (EB, 8))
    bytes_rep = jnp.concatenate([b0, b1], axis=1)
    col = lax.broadcasted_iota(jnp.int32, (1, 16), 1)
    shift = 7 - (col % 8)
    pw = (jnp.int32(1) << shift).astype(jnp.float32)
    t = jnp.floor(bytes_rep / pw)
    return t - 2.0 * jnp.floor(t * 0.5)


def _rep_matrix(in_c, out_c):
    r = lax.broadcasted_iota(jnp.int32, (in_c, in_c * out_c), 0)
    c = lax.broadcasted_iota(jnp.int32, (in_c, in_c * out_c), 1)
    return (c // out_c == r).astype(jnp.float32)


def _sum_matrix(in_c, out_c):
    r = lax.broadcasted_iota(jnp.int32, (in_c * out_c, out_c), 0)
    c = lax.broadcasted_iota(jnp.int32, (in_c * out_c, out_c), 1)
    return (r % out_c == c).astype(jnp.float32)


def _dot(a, b):
    return jax.lax.dot_general(a, b, (((1,), (0,)), ((), ())),
                               preferred_element_type=jnp.float32)


def _msg_body(in_c, out_c, ea_ref, xs_ref, w1_ref, b1_ref, w2_ref, b2_ref,
              out_ref):
    bits = _unpack_bits_block(ea_ref[...])
    h1 = jnp.maximum(_dot(bits, w1_ref[...]) + b1_ref[...], 0.0)
    we = _dot(h1, w2_ref[...]) + b2_ref[...]
    xs_rep = _dot(xs_ref[...], _rep_matrix(in_c, out_c))
    out_ref[...] = _dot(we * xs_rep, _sum_matrix(in_c, out_c))


def _msg_call(in_c, out_c, hid_dim, eap, xsp, W1, b1, W2, b2):
    body = functools.partial(_msg_body, in_c, out_c)
    return pl.pallas_call(
        body,
        grid=(N_EBLK,),
        in_specs=[
            pl.BlockSpec((EB, 2), lambda i: (i, 0)),
            pl.BlockSpec((EB, in_c), lambda i: (i, 0)),
            pl.BlockSpec((16, hid_dim), lambda i: (0, 0)),
            pl.BlockSpec((1, hid_dim), lambda i: (0, 0)),
            pl.BlockSpec((hid_dim, in_c * out_c), lambda i: (0, 0)),
            pl.BlockSpec((1, in_c * out_c), lambda i: (0, 0)),
        ],
        out_specs=pl.BlockSpec((EB, out_c), lambda i: (i, 0)),
        out_shape=jax.ShapeDtypeStruct((EP, out_c), jnp.float32),
    )(eap, xsp, W1, b1.reshape(1, -1), W2, b2.reshape(1, -1))


def _node1_body(msgp_ref, cntp_ref, x_ref, root_ref, bias_ref,
                h_ref, cm_ref):
    s = msgp_ref[0] + msgp_ref[1]
    cnt = cntp_ref[0, :, 0:1] + cntp_ref[1, :, 0:1]
    cm = jnp.maximum(cnt, 1.0)
    h = s / cm + _dot(x_ref[...], root_ref[...]) + bias_ref[...]
    h_ref[...] = jnp.maximum(h, 0.0)
    cm_ref[...] = cm


def _node1_call(msgp, cntp, x, root1, bias1):
    return pl.pallas_call(
        _node1_body,
        out_shape=[jax.ShapeDtypeStruct((N, HID), jnp.float32),
                   jax.ShapeDtypeStruct((N, 1), jnp.float32)],
    )(msgp, cntp, x, root1, bias1.reshape(1, -1))


def _final_body(s2p_ref, cm_ref, h_ref, root_ref, bias_ref, batch_ref,
                fc1w, fc1b, fc2w, fc2b, fc3w, fc3b, fc4w, fc4b, out_ref):
    s2 = s2p_ref[0] + s2p_ref[1]
    h2 = s2 / cm_ref[...] + _dot(h_ref[...], root_ref[...]) + bias_ref[...]
    h2 = jnp.maximum(h2, 0.0)
    gi = lax.broadcasted_iota(jnp.int32, (N_GRAPHS, N), 0)
    oh = (gi == batch_ref[...]).astype(jnp.float32)
    pooled = _dot(oh, h2)
    gc = jnp.sum(oh, axis=1, keepdims=True)
    g = pooled / jnp.maximum(gc, 1.0)
    g = jnp.maximum(_dot(g, fc1w[...]) + fc1b[...], 0.0)
    g = jnp.maximum(_dot(g, fc2w[...]) + fc2b[...], 0.0)
    g = jnp.maximum(_dot(g, fc3w[...]) + fc3b[...], 0.0)
    out_ref[...] = _dot(g, fc4w[...]) + fc4b[...]


def _final_call(s2p, cm, h, root2, bias2, batch2d,
                fc1_W, fc1_b, fc2_W, fc2_b, fc3_W, fc3_b, fc4_W, fc4_b):
    return pl.pallas_call(
        _final_body,
        out_shape=jax.ShapeDtypeStruct((N_GRAPHS, N_CLASSES), jnp.float32),
    )(s2p, cm, h, root2, bias2.reshape(1, -1), batch2d,
      fc1_W, fc1_b.reshape(1, -1), fc2_W, fc2_b.reshape(1, -1),
      fc3_W, fc3_b.reshape(1, -1), fc4_W, fc4_b.reshape(1, -1))


def kernel(x, edge_index, edge_attr, batch, en1_W1, en1_b1, en1_W2, en1_b2,
           en2_W1, en2_b1, en2_W2, en2_b2, root1, bias1, root2, bias2,
           fc1_W, fc1_b, fc2_W, fc2_b, fc3_W, fc3_b, fc4_W, fc4_b):
    src, dst = edge_index[0], edge_index[1]
    pad = EP - E
    src2d = jnp.concatenate([src, jnp.zeros((pad,), jnp.int32)]).reshape(-1, CHUNK)
    dst2d = jnp.concatenate([dst, jnp.full((pad,), N, jnp.int32)]).reshape(-1, CHUNK)
    eap = jnp.concatenate([edge_attr, jnp.zeros((pad, 2), jnp.int32)], axis=0)

    xs = _make_gather(IN_C)(x, src2d)
    msg1 = _msg_call(IN_C, HID, HID * IN_C, eap, xs,
                     en1_W1, en1_b1, en1_W2, en1_b2)
    msgp, cntp = _make_scatter1()(msg1, dst2d)
    h, cm = _node1_call(msgp, cntp, x, root1, bias1)

    hs = _make_gather(HID)(h, src2d)
    msg2 = _msg_call(HID, 2 * HID, HID * HID, eap, hs,
                     en2_W1, en2_b1, en2_W2, en2_b2)
    s2p = _make_scatter2()(msg2, dst2d)

    return _final_call(s2p, cm, h, root2, bias2, batch.reshape(1, N),
                       fc1_W, fc1_b, fc2_W, fc2_b, fc3_W, fc3_b, fc4_W, fc4_b)


# speedup vs baseline: 2.2716x; 1.0548x over previous
import functools

import jax
import jax.numpy as jnp
from jax import lax
from jax.experimental import pallas as pl
from jax.experimental.pallas import tpu as pltpu
from jax.experimental.pallas import tpu_sc as plsc

N = 10000
E = 160000
IN_C = 32
HID = 16
N_CLASSES = 10
N_GRAPHS = 64

NC = 2
NS = 16
CHUNK = 128
CHUNKS_PER_TILE = 40
GROUP = 4
NGROUPS = CHUNKS_PER_TILE // GROUP
GCHUNK = GROUP * CHUNK
E_TILE = CHUNK * CHUNKS_PER_TILE
EP = E_TILE * NC * NS
NROWS = N + 16
ZROWS = NROWS // NS
OROWS = N // NS

EB = 640
N_EBLK = EP // EB


def _mesh():
    return plsc.VectorSubcoreMesh(core_axis_name="c", subcore_axis_name="s")


_SC_PARAMS = pltpu.CompilerParams(use_tc_tiling_on_sc=False)


def _gather_body(d, table_hbm, idx2d_hbm, out_hbm, idx_v, rows_v, sem):
    c = lax.axis_index("c")
    s = lax.axis_index("s")
    wid = s * NC + c
    pltpu.sync_copy(idx2d_hbm.at[pl.ds(wid * CHUNKS_PER_TILE, CHUNKS_PER_TILE)],
                    idx_v)

    def body(g, carry):
        descs = [
            pltpu.async_copy(table_hbm.at[idx_v.at[g * GROUP + b]],
                             rows_v.at[pl.ds(b * CHUNK, CHUNK)], sem)
            for b in range(GROUP)
        ]
        for dd in descs:
            dd.wait()
        pltpu.sync_copy(rows_v,
                        out_hbm.at[pl.ds(wid * E_TILE + g * GCHUNK, GCHUNK)])
        return carry

    lax.fori_loop(0, NGROUPS, body, 0)


def _make_gather(d):
    return functools.partial(
        pl.kernel,
        out_type=jax.ShapeDtypeStruct((EP, d), jnp.float32),
        mesh=_mesh(),
        compiler_params=_SC_PARAMS,
        scratch_types=[
            pltpu.VMEM((CHUNKS_PER_TILE, CHUNK), jnp.int32),
            pltpu.VMEM((GCHUNK, d), jnp.float32),
            pltpu.SemaphoreType.DMA,
        ],
    )(functools.partial(_gather_body, d))


def _scatter1_body(msg_hbm, idx2d_hbm, msg_out, cnt_out,
                   idx_v, msg_buf, ones_buf, zrow, tab_msg, tab_cnt, sem):
    c = lax.axis_index("c")
    s = lax.axis_index("s")
    wid = s * NC + c

    z16 = jnp.zeros((16,), jnp.float32)
    o16 = jnp.ones((16,), jnp.float32)

    def fill_z(i, carry):
        zrow[i, :] = z16
        return carry

    lax.fori_loop(0, ZROWS, fill_z, 0)

    def fill_o(i, carry):
        ones_buf[i, :] = o16
        return carry

    lax.fori_loop(0, CHUNK, fill_o, 0)

    pltpu.sync_copy(zrow, tab_msg.at[pl.ds(s * ZROWS, ZROWS)])
    pltpu.sync_copy(zrow, tab_cnt.at[pl.ds(s * ZROWS, ZROWS)])
    plsc.subcore_barrier()

    pltpu.sync_copy(idx2d_hbm.at[pl.ds(wid * CHUNKS_PER_TILE, CHUNKS_PER_TILE)],
                    idx_v)

    def body(g, carry):
        pltpu.sync_copy(msg_hbm.at[pl.ds(wid * E_TILE + g * GCHUNK, GCHUNK)],
                        msg_buf)
        descs = []
        for b in range(GROUP):
            idx_row = idx_v.at[g * GROUP + b]
            descs.append(pltpu.async_copy(
                msg_buf.at[pl.ds(b * CHUNK, CHUNK)], tab_msg.at[idx_row],
                sem, add=True))
            descs.append(pltpu.async_copy(
                ones_buf, tab_cnt.at[idx_row], sem, add=True))
        for dd in descs:
            dd.wait()
        return carry

    lax.fori_loop(0, NGROUPS, body, 0)
    plsc.subcore_barrier()

    pltpu.sync_copy(tab_msg.at[pl.ds(s * OROWS, OROWS)],
                    msg_out.at[c, pl.ds(s * OROWS, OROWS)])
    pltpu.sync_copy(tab_cnt.at[pl.ds(s * OROWS, OROWS)],
                    cnt_out.at[c, pl.ds(s * OROWS, OROWS)])


def _make_scatter1():
    return functools.partial(
        pl.kernel,
        out_type=[jax.ShapeDtypeStruct((NC, N, HID), jnp.float32),
                  jax.ShapeDtypeStruct((NC, N, HID), jnp.float32)],
        mesh=_mesh(),
        compiler_params=_SC_PARAMS,
        scratch_types=[
            pltpu.VMEM((CHUNKS_PER_TILE, CHUNK), jnp.int32),
            pltpu.VMEM((GCHUNK, HID), jnp.float32),
            pltpu.VMEM((CHUNK, HID), jnp.float32),
            pltpu.VMEM((ZROWS, HID), jnp.float32),
            pltpu.VMEM_SHARED((NROWS, HID), jnp.float32),
            pltpu.VMEM_SHARED((NROWS, HID), jnp.float32),
            pltpu.SemaphoreType.DMA,
        ],
    )(_scatter1_body)


def _scatter2_body(msg_hbm, idx2d_hbm, msg_out, idx_v, msg_buf, zrow, tab_msg,
                   sem):
    c = lax.axis_index("c")
    s = lax.axis_index("s")
    wid = s * NC + c

    z16 = jnp.zeros((16,), jnp.float32)

    def fill_z(i, carry):
        zrow[i, 0:16] = z16
        zrow[i, 16:32] = z16
        return carry

    lax.fori_loop(0, ZROWS, fill_z, 0)

    pltpu.sync_copy(zrow, tab_msg.at[pl.ds(s * ZROWS, ZROWS)])
    plsc.subcore_barrier()

    pltpu.sync_copy(idx2d_hbm.at[pl.ds(wid * CHUNKS_PER_TILE, CHUNKS_PER_TILE)],
                    idx_v)

    def body(g, carry):
        pltpu.sync_copy(msg_hbm.at[pl.ds(wid * E_TILE + g * GCHUNK, GCHUNK)],
                        msg_buf)
        descs = [
            pltpu.async_copy(msg_buf.at[pl.ds(b * CHUNK, CHUNK)],
                             tab_msg.at[idx_v.at[g * GROUP + b]], sem, add=True)
            for b in range(GROUP)
        ]
        for dd in descs:
            dd.wait()
        return carry

    lax.fori_loop(0, NGROUPS, body, 0)
    plsc.subcore_barrier()

    pltpu.sync_copy(tab_msg.at[pl.ds(s * OROWS, OROWS)],
                    msg_out.at[c, pl.ds(s * OROWS, OROWS)])


def _make_scatter2():
    return functools.partial(
        pl.kernel,
        out_type=jax.ShapeDtypeStruct((NC, N, 2 * HID), jnp.float32),
        mesh=_mesh(),
        compiler_params=_SC_PARAMS,
        scratch_types=[
            pltpu.VMEM((CHUNKS_PER_TILE, CHUNK), jnp.int32),
            pltpu.VMEM((GCHUNK, 2 * HID), jnp.float32),
            pltpu.VMEM((ZROWS, 2 * HID), jnp.float32),
            pltpu.VMEM_SHARED((NROWS, 2 * HID), jnp.float32),
            pltpu.SemaphoreType.DMA,
        ],
    )(_scatter2_body)


def _unpack_bits_block(ea_i32):
    ea_f = ea_i32.astype(jnp.float32)
    b0 = jnp.broadcast_to(ea_f[:, 0:1], (EB, 8))
    b1 = jnp.broadcast_to(ea_f[:, 1:2], (EB, 8))
    bytes_rep = jnp.concatenate([b0, b1], axis=1)
    col = lax.broadcasted_iota(jnp.int32, (1, 16), 1)
    shift = 7 - (col % 8)
    pw = (jnp.int32(1) << shift).astype(jnp.float32)
    t = jnp.floor(bytes_rep / pw)
    return t - 2.0 * jnp.floor(t * 0.5)


def _rep_matrix(in_c, out_c):
    r = lax.broadcasted_iota(jnp.int32, (in_c, in_c * out_c), 0)
    c = lax.broadcasted_iota(jnp.int32, (in_c, in_c * out_c), 1)
    return (c // out_c == r).astype(jnp.float32)


def _sum_matrix(in_c, out_c):
    r = lax.broadcasted_iota(jnp.int32, (in_c * out_c, out_c), 0)
    c = lax.broadcasted_iota(jnp.int32, (in_c * out_c, out_c), 1)
    return (r % out_c == c).astype(jnp.float32)


def _dot(a, b):
    return jax.lax.dot_general(a, b, (((1,), (0,)), ((), ())),
                               preferred_element_type=jnp.float32)


def _msg_body(in_c, out_c, ea_ref, xs_ref, w1_ref, b1_ref, w2_ref, b2_ref,
              out_ref):
    bits = _unpack_bits_block(ea_ref[...])
    h1 = jnp.maximum(_dot(bits, w1_ref[...]) + b1_ref[...], 0.0)
    we = _dot(h1, w2_ref[...]) + b2_ref[...]
    xs_rep = _dot(xs_ref[...], _rep_matrix(in_c, out_c))
    out_ref[...] = _dot(we * xs_rep, _sum_matrix(in_c, out_c))


def _msg_call(in_c, out_c, hid_dim, eap, xsp, W1, b1, W2, b2):
    body = functools.partial(_msg_body, in_c, out_c)
    return pl.pallas_call(
        body,
        grid=(N_EBLK,),
        in_specs=[
            pl.BlockSpec((EB, 2), lambda i: (i, 0)),
            pl.BlockSpec((EB, in_c), lambda i: (i, 0)),
            pl.BlockSpec((16, hid_dim), lambda i: (0, 0)),
            pl.BlockSpec((1, hid_dim), lambda i: (0, 0)),
            pl.BlockSpec((hid_dim, in_c * out_c), lambda i: (0, 0)),
            pl.BlockSpec((1, in_c * out_c), lambda i: (0, 0)),
        ],
        out_specs=pl.BlockSpec((EB, out_c), lambda i: (i, 0)),
        out_shape=jax.ShapeDtypeStruct((EP, out_c), jnp.float32),
    )(eap, xsp, W1, b1.reshape(1, -1), W2, b2.reshape(1, -1))


def _node1_body(msgp_ref, cntp_ref, x_ref, root_ref, bias_ref,
                h_ref, cm_ref):
    s = msgp_ref[0] + msgp_ref[1]
    cnt = cntp_ref[0, :, 0:1] + cntp_ref[1, :, 0:1]
    cm = jnp.maximum(cnt, 1.0)
    h = s / cm + _dot(x_ref[...], root_ref[...]) + bias_ref[...]
    h_ref[...] = jnp.maximum(h, 0.0)
    cm_ref[...] = cm


def _node1_call(msgp, cntp, x, root1, bias1):
    return pl.pallas_call(
        _node1_body,
        out_shape=[jax.ShapeDtypeStruct((N, HID), jnp.float32),
                   jax.ShapeDtypeStruct((N, 1), jnp.float32)],
    )(msgp, cntp, x, root1, bias1.reshape(1, -1))


def _final_body(s2p_ref, cm_ref, h_ref, root_ref, bias_ref, batch_ref,
                fc1w, fc1b, fc2w, fc2b, fc3w, fc3b, fc4w, fc4b, out_ref):
    s2 = s2p_ref[0] + s2p_ref[1]
    h2 = s2 / cm_ref[...] + _dot(h_ref[...], root_ref[...]) + bias_ref[...]
    h2 = jnp.maximum(h2, 0.0)
    gi = lax.broadcasted_iota(jnp.int32, (N_GRAPHS, N), 0)
    oh = (gi == batch_ref[...]).astype(jnp.float32)
    pooled = _dot(oh, h2)
    gc = jnp.sum(oh, axis=1, keepdims=True)
    g = pooled / jnp.maximum(gc, 1.0)
    g = jnp.maximum(_dot(g, fc1w[...]) + fc1b[...], 0.0)
    g = jnp.maximum(_dot(g, fc2w[...]) + fc2b[...], 0.0)
    g = jnp.maximum(_dot(g, fc3w[...]) + fc3b[...], 0.0)
    out_ref[...] = _dot(g, fc4w[...]) + fc4b[...]


def _final_call(s2p, cm, h, root2, bias2, batch2d,
                fc1_W, fc1_b, fc2_W, fc2_b, fc3_W, fc3_b, fc4_W, fc4_b):
    return pl.pallas_call(
        _final_body,
        out_shape=jax.ShapeDtypeStruct((N_GRAPHS, N_CLASSES), jnp.float32),
    )(s2p, cm, h, root2, bias2.reshape(1, -1), batch2d,
      fc1_W, fc1_b.reshape(1, -1), fc2_W, fc2_b.reshape(1, -1),
      fc3_W, fc3_b.reshape(1, -1), fc4_W, fc4_b.reshape(1, -1))


def kernel(x, edge_index, edge_attr, batch, en1_W1, en1_b1, en1_W2, en1_b2,
           en2_W1, en2_b1, en2_W2, en2_b2, root1, bias1, root2, bias2,
           fc1_W, fc1_b, fc2_W, fc2_b, fc3_W, fc3_b, fc4_W, fc4_b):
    src, dst = edge_index[0], edge_index[1]
    pad = EP - E
    src2d = jnp.concatenate([src, jnp.zeros((pad,), jnp.int32)]).reshape(-1, CHUNK)
    dst2d = jnp.concatenate([dst, jnp.full((pad,), N, jnp.int32)]).reshape(-1, CHUNK)
    eap = jnp.concatenate([edge_attr, jnp.zeros((pad, 2), jnp.int32)], axis=0)

    xs = _make_gather(IN_C)(x, src2d)
    msg1 = _msg_call(IN_C, HID, HID * IN_C, eap, xs,
                     en1_W1, en1_b1, en1_W2, en1_b2)
    msgp, cntp = _make_scatter1()(msg1, dst2d)
    h, cm = _node1_call(msgp, cntp, x, root1, bias1)

    hs = _make_gather(HID)(h, src2d)
    msg2 = _msg_call(HID, 2 * HID, HID * HID, eap, hs,
                     en2_W1, en2_b1, en2_W2, en2_b2)
    s2p = _make_scatter2()(msg2, dst2d)

    return _final_call(s2p, cm, h, root2, bias2, batch.reshape(1, N),
                       fc1_W, fc1_b, fc2_W, fc2_b, fc3_W, fc3_b, fc4_W, fc4_b)
